# baseline (speedup 1.0000x reference)
import jax
import jax.numpy as jnp
from jax import lax
from jax.experimental import pallas as pl
from jax.experimental.pallas import tpu as pltpu
from jax.experimental.pallas import tpu_sc as plsc

B = 8
L = 4096
D = 1024
T = 256
K = L // T
TB = 1024
KO = L // TB
JJ = TB // T
WIN = 272
CLIP_LO = 1e-4
CLIP_HI = 1.0 - 1e-4


def _prep_body(mask_hbm, p_hbm, psort_hbm, idx_hbm, nb_hbm,
               mask_v, p_v, psort_v, idx_v, sem):
    wid = lax.axis_index("s") * 2 + lax.axis_index("c")

    @pl.when(wid < B)
    def _():
        pltpu.async_copy(mask_hbm.at[wid], mask_v, sem).wait()
        pltpu.async_copy(p_hbm.at[wid], p_v, sem).wait()

        fill = jnp.full((16,), 0.5, jnp.float32)

        def body(j, cnt_vec):
            m16 = mask_v[pl.ds(j * 16, 16)]
            mb = m16 > 0
            p16 = p_v[pl.ds(j * 16, 16)]
            psort_v[pl.ds(j * 16, 16)] = fill
            cum = plsc.cumsum(m16)
            pbv = cum + cnt_vec - 1
            idx_v[pl.ds(j * 16, 16)] = pbv
            plsc.store_scatter(psort_v, [pbv], p16, mask=mb)
            return cnt_vec + plsc.all_reduce_population_count(mb)

        total = lax.fori_loop(0, L // 16, body, jnp.zeros((16,), jnp.int32))

        pltpu.async_copy(psort_v, psort_hbm.at[wid], sem).wait()
        pltpu.async_copy(idx_v, idx_hbm.at[wid], sem).wait()
        mask_v[pl.ds(0, 16)] = total
        pltpu.async_copy(mask_v.at[pl.ds(0, 16)],
                         nb_hbm.at[pl.ds(wid * 16, 16)], sem).wait()


def _sc_prep(mask_i32, p_raw):
    mesh = plsc.VectorSubcoreMesh(core_axis_name="c", subcore_axis_name="s")
    fn = pl.kernel(
        _prep_body,
        mesh=mesh,
        out_type=(
            jax.ShapeDtypeStruct((B, L), jnp.float32),
            jax.ShapeDtypeStruct((B, L), jnp.int32),
            jax.ShapeDtypeStruct((B * 16,), jnp.int32),
        ),
        scratch_types=[
            pltpu.VMEM((L,), jnp.int32),
            pltpu.VMEM((L,), jnp.float32),
            pltpu.VMEM((L,), jnp.float32),
            pltpu.VMEM((L,), jnp.int32),
            pltpu.SemaphoreType.DMA,
        ],
        compiler_params=pltpu.CompilerParams(needs_layout_passes=False),
    )
    return fn(mask_i32, p_raw)


def _scan_body(nb_ref, ps_ref, hs_ref, idx_ref, out_ref, h_ref, yscr_ref):
    b = pl.program_id(0)
    ko = pl.program_id(1)
    nb = nb_ref[b]

    @pl.when(ko == 0)
    def _():
        h_ref[...] = jnp.zeros_like(h_ref)
        yscr_ref[pl.ds(T, WIN - T), :] = jnp.zeros((WIN - T, D), jnp.bfloat16)

    row = lax.broadcasted_iota(jnp.int32, (T, T), 0)
    col = lax.broadcasted_iota(jnp.int32, (T, T), 1)
    tri_u = (row <= col).astype(jnp.bfloat16)

    for j in range(JJ):
        kg = ko * JJ + j
        valid = kg * T < nb

        @pl.when(valid)
        def _(j=j, kg=kg):
            ps = ps_ref[0, j, :, :]
            p = jnp.clip(ps, CLIP_LO, CLIP_HI)
            dt = jnp.log(1.0 / (1.0 - p))
            dth = dt.astype(jnp.bfloat16).astype(jnp.float32)
            pb = p.astype(jnp.bfloat16).astype(jnp.float32)

            dt_bc = jnp.broadcast_to(dth, (T, T))
            s_col = jnp.sum(jnp.where(col <= row, dt_bc, 0.0),
                            axis=1, keepdims=True)
            s_row = jnp.dot(dth.astype(jnp.bfloat16), tri_u,
                            preferred_element_type=jnp.float32)

            mlog = jnp.broadcast_to(s_row, (T, T)) - s_col
            w = (jnp.where(col <= row, jnp.exp(mlog), 0.0)
                 * jnp.broadcast_to(pb, (T, T)))

            xs = hs_ref[0, pl.ds(j * T, T), :]
            xb = xs.astype(jnp.bfloat16)

            y0 = jnp.dot(w.astype(jnp.bfloat16), xb,
                         preferred_element_type=jnp.float32)
            y = y0 + jnp.exp(-s_col) * h_ref[0:1, :]
            h_ref[0:1, :] = y[T - 1:T, :]
            yscr_ref[pl.ds(kg * T, T), :] = y.astype(jnp.bfloat16)

        @pl.when(jnp.logical_not(valid) & ((kg - 2) * T < nb))
        def _(kg=kg):
            yscr_ref[pl.ds(kg * T, T), :] = jnp.zeros((T, D), jnp.bfloat16)

        pbcol = idx_ref[0, j, :, :]
        w0 = idx_ref[0, j, 0, 0]
        w0a = pl.multiple_of(
            jnp.maximum(jnp.minimum((w0 // 16) * 16, (kg + 1) * T - WIN), 0),
            16)
        ywin = yscr_ref[pl.ds(w0a, WIN), :]
        local = pbcol - w0a
        oh = (jnp.broadcast_to(local, (T, WIN))
              == lax.broadcasted_iota(jnp.int32, (T, WIN), 1)
              ).astype(jnp.bfloat16)
        out_ref[0, pl.ds(j * T, T), :] = jnp.dot(
            oh, ywin, preferred_element_type=jnp.float32)


def _tc_scan_gather(p_sorted, hidden, idx, nb):
    ps_r = p_sorted.reshape(B, K, 1, T)
    idx_r = idx.reshape(B, K, T, 1)

    def last_blk(nb_ref, b):
        return jnp.minimum(jnp.maximum(nb_ref[b] - 1, 0) // TB, KO - 1)

    grid_spec = pltpu.PrefetchScalarGridSpec(
        num_scalar_prefetch=1,
        grid=(B, KO),
        in_specs=[
            pl.BlockSpec((1, JJ, 1, T),
                         lambda b, ko, nb_ref:
                         (b, jnp.minimum(ko, last_blk(nb_ref, b)), 0, 0)),
            pl.BlockSpec((1, TB, D),
                         lambda b, ko, nb_ref:
                         (b, jnp.minimum(ko, last_blk(nb_ref, b)), 0)),
            pl.BlockSpec((1, JJ, T, 1),
                         lambda b, ko, nb_ref: (b, ko, 0, 0)),
        ],
        out_specs=pl.BlockSpec((1, TB, D), lambda b, ko, nb_ref: (b, ko, 0)),
        scratch_shapes=[
            pltpu.VMEM((8, D), jnp.float32),
            pltpu.VMEM((L, D), jnp.bfloat16),
        ],
    )
    return pl.pallas_call(
        _scan_body,
        grid_spec=grid_spec,
        out_shape=jax.ShapeDtypeStruct((B, L, D), jnp.float32),
        compiler_params=pltpu.CompilerParams(
            dimension_semantics=("parallel", "arbitrary"),
        ),
    )(nb, ps_r, hidden, idx_r)


def kernel(hidden_states, boundary_mask, boundary_prob):
    mask_i32 = boundary_mask.astype(jnp.int32)
    p_raw = boundary_prob[..., 1].astype(jnp.float32)
    p_sorted, idx, nb16 = _sc_prep(mask_i32, p_raw)
    nb = nb16.reshape(B, 16)[:, 0]
    return _tc_scan_gather(p_sorted, hidden_states, idx, nb)
